# Initial kernel scaffold; baseline (speedup 1.0000x reference)
import jax
import jax.numpy as jnp
from jax.experimental import pallas as pl


def kernel(X, table, pe):
    raise NotImplementedError("write your pallas kernel here")


# speedup vs baseline: 2.2793x; 2.2793x over previous
import functools

import jax
import jax.numpy as jnp
from jax import lax
from jax.experimental import pallas as pl
from jax.experimental.pallas import tpu as pltpu
from jax.experimental.pallas import tpu_sc as plsc

NC = 2
NS = 16
NW = NC * NS
LANES = 16


def kernel(X, table, pe):
    B, T = X.shape
    V, D = table.shape
    N = B * T
    scale_val = float(D) ** 0.5

    n_per_w = N // NW
    C = 400
    n_chunks = n_per_w // C

    xflat = X.reshape(N)
    pe_t = pe[:T]

    mesh = plsc.VectorSubcoreMesh(
        core_axis_name="c", subcore_axis_name="s",
        num_cores=NC, num_subcores=NS)

    @functools.partial(
        pl.kernel,
        out_type=jax.ShapeDtypeStruct((N, D), jnp.float32),
        mesh=mesh,
        scratch_types=[
            pltpu.VMEM((C,), jnp.int32),
            pltpu.VMEM((C, D), jnp.float32),
            pltpu.VMEM((T, D), jnp.float32),
            pltpu.SemaphoreType.DMA,
        ],
        compiler_params=pltpu.CompilerParams(
            needs_layout_passes=False, use_tc_tiling_on_sc=False),
    )
    def emb_kernel(x_hbm, tbl_hbm, pe_hbm, out_hbm, idx_v, rows_v, pe_v, sem):
        wid = lax.axis_index("s") * NC + lax.axis_index("c")
        base = wid * n_per_w
        pltpu.sync_copy(pe_hbm, pe_v)

        def chunk_body(k, carry):
            cbase = base + k * C
            pltpu.sync_copy(x_hbm.at[pl.ds(cbase, C)], idx_v)
            pltpu.async_copy(tbl_hbm.at[idx_v], rows_v, sem).wait()

            def tok_body(j, carry2):
                prow = lax.rem(j, T)
                idx_splat = plsc.load_gather(
                    idx_v, [jnp.full((LANES,), j, jnp.int32)])
                s = jnp.where(idx_splat != 0,
                              jnp.float32(scale_val), jnp.float32(0.0))
                for d in range(D // LANES):
                    sl = pl.ds(d * LANES, LANES)
                    rows_v[j, sl] = rows_v[j, sl] * s + pe_v[prow, sl]
                return carry2

            lax.fori_loop(0, C, tok_body, None, unroll=4)
            pltpu.sync_copy(rows_v, out_hbm.at[pl.ds(cbase, C)])
            return carry

        lax.fori_loop(0, n_chunks, chunk_body, None)

    out = emb_kernel(xflat, table, pe_t)
    return out.reshape(B, T, D)


# speedup vs baseline: 2.5156x; 1.1037x over previous
import functools

import jax
import jax.numpy as jnp
from jax import lax
from jax.experimental import pallas as pl
from jax.experimental.pallas import tpu as pltpu
from jax.experimental.pallas import tpu_sc as plsc

NC = 2
NS = 16
NW = NC * NS
LANES = 16
NB = 4


def kernel(X, table, pe):
    B, T = X.shape
    V, D = table.shape
    N = B * T
    scale_val = float(D) ** 0.5

    n_per_w = N // NW
    C = 400
    n_batches = n_per_w // (C * NB)

    xflat = X.reshape(N)
    pe_t = pe[:T]

    mesh = plsc.VectorSubcoreMesh(
        core_axis_name="c", subcore_axis_name="s",
        num_cores=NC, num_subcores=NS)

    @functools.partial(
        pl.kernel,
        out_type=jax.ShapeDtypeStruct((N, D), jnp.float32),
        mesh=mesh,
        scratch_types=[
            pltpu.VMEM((NB * C,), jnp.int32),
            pltpu.VMEM((NB, C, D), jnp.float32),
            pltpu.VMEM((T, D), jnp.float32),
            pltpu.SemaphoreType.DMA((NB,)),
            pltpu.SemaphoreType.DMA((NB,)),
        ],
        compiler_params=pltpu.CompilerParams(
            needs_layout_passes=False, use_tc_tiling_on_sc=False),
    )
    def emb_kernel(x_hbm, tbl_hbm, pe_hbm, out_hbm,
                   idx_v, rows_v, pe_v, gsem, ssem):
        wid = lax.axis_index("s") * NC + lax.axis_index("c")
        base = wid * n_per_w
        pltpu.sync_copy(pe_hbm, pe_v)

        def batch_body(g, carry):
            bbase = base + g * (NB * C)
            pltpu.sync_copy(x_hbm.at[pl.ds(bbase, NB * C)], idx_v)

            for i in range(NB):
                @pl.when(g > 0)
                def _():
                    pltpu.make_async_copy(
                        rows_v.at[i], out_hbm.at[pl.ds(bbase, C)],
                        ssem.at[i]).wait()
                pltpu.make_async_copy(
                    tbl_hbm.at[idx_v.at[pl.ds(i * C, C)]],
                    rows_v.at[i], gsem.at[i]).start()

            for i in range(NB):
                cbase = bbase + i * C
                pltpu.make_async_copy(
                    tbl_hbm.at[idx_v.at[pl.ds(i * C, C)]],
                    rows_v.at[i], gsem.at[i]).wait()

                def tok_body(j, carry2):
                    prow = lax.rem(cbase - base + j, T)
                    idx_splat = plsc.load_gather(
                        idx_v, [jnp.full((LANES,), i * C + j, jnp.int32)])
                    s = jnp.where(idx_splat != 0,
                                  jnp.float32(scale_val), jnp.float32(0.0))
                    for d in range(D // LANES):
                        sl = pl.ds(d * LANES, LANES)
                        rows_v[i, j, sl] = rows_v[i, j, sl] * s + pe_v[prow, sl]
                    return carry2

                lax.fori_loop(0, C, tok_body, None, unroll=4)
                pltpu.make_async_copy(
                    rows_v.at[i], out_hbm.at[pl.ds(cbase, C)],
                    ssem.at[i]).start()
            return carry

        lax.fori_loop(0, n_batches, batch_body, None)

        last = base + (n_batches - 1) * (NB * C)
        for i in range(NB):
            pltpu.make_async_copy(
                rows_v.at[i], out_hbm.at[pl.ds(last + i * C, C)],
                ssem.at[i]).wait()

    out = emb_kernel(xflat, table, pe_t)
    return out.reshape(B, T, D)


# speedup vs baseline: 3.7812x; 1.5031x over previous
import functools

import jax
import jax.numpy as jnp
from jax import lax
from jax.experimental import pallas as pl
from jax.experimental.pallas import tpu as pltpu
from jax.experimental.pallas import tpu_sc as plsc

NC = 2
NS = 16
NW = NC * NS
LANES = 16
NB = 4


def kernel(X, table, pe):
    B, T = X.shape
    V, D = table.shape
    N = B * T
    scale_val = float(D) ** 0.5

    n_per_w = N // NW
    C = 400
    n_batches = n_per_w // (C * NB)

    xflat = X.reshape(N)
    pe_t = pe[:T]

    mesh = plsc.VectorSubcoreMesh(
        core_axis_name="c", subcore_axis_name="s",
        num_cores=NC, num_subcores=NS)

    @functools.partial(
        pl.kernel,
        out_type=jax.ShapeDtypeStruct((N, D), jnp.float32),
        mesh=mesh,
        scratch_types=[
            pltpu.VMEM((NB * C,), jnp.int32),
            pltpu.VMEM((NB, C, D), jnp.float32),
            pltpu.VMEM((T, D), jnp.float32),
            pltpu.SemaphoreType.DMA((NB,)),
            pltpu.SemaphoreType.DMA((NB,)),
        ],
        compiler_params=pltpu.CompilerParams(
            needs_layout_passes=False, use_tc_tiling_on_sc=False),
    )
    def emb_kernel(x_hbm, tbl_hbm, pe_hbm, out_hbm,
                   idx_v, rows_v, pe_v, gsem, ssem):
        wid = lax.axis_index("s") * NC + lax.axis_index("c")
        base = wid * n_per_w
        pltpu.sync_copy(pe_hbm, pe_v)

        def batch_body(g, carry):
            bbase = base + g * (NB * C)
            pltpu.sync_copy(x_hbm.at[pl.ds(bbase, NB * C)], idx_v)

            for i in range(NB):
                @pl.when(g > 0)
                def _():
                    pltpu.make_async_copy(
                        rows_v.at[i], out_hbm.at[pl.ds(bbase, C)],
                        ssem.at[i]).wait()
                pltpu.make_async_copy(
                    tbl_hbm.at[idx_v.at[pl.ds(i * C, C)]],
                    rows_v.at[i], gsem.at[i]).start()

            for i in range(NB):
                cbase = bbase + i * C
                pltpu.make_async_copy(
                    tbl_hbm.at[idx_v.at[pl.ds(i * C, C)]],
                    rows_v.at[i], gsem.at[i]).wait()

                @plsc.parallel_loop(0, C, 1, unroll=8)
                def tok_body(j):
                    prow = lax.rem(cbase - base + j, T)
                    idx_splat = plsc.load_gather(
                        idx_v, [jnp.full((LANES,), i * C + j, jnp.int32)])
                    s = jnp.where(idx_splat != 0,
                                  jnp.float32(scale_val), jnp.float32(0.0))
                    for d in range(D // LANES):
                        sl = pl.ds(d * LANES, LANES)
                        rows_v[i, j, sl] = rows_v[i, j, sl] * s + pe_v[prow, sl]
                pltpu.make_async_copy(
                    rows_v.at[i], out_hbm.at[pl.ds(cbase, C)],
                    ssem.at[i]).start()
            return carry

        lax.fori_loop(0, n_batches, batch_body, None)

        last = base + (n_batches - 1) * (NB * C)
        for i in range(NB):
            pltpu.make_async_copy(
                rows_v.at[i], out_hbm.at[pl.ds(last + i * C, C)],
                ssem.at[i]).wait()

    out = emb_kernel(xflat, table, pe_t)
    return out.reshape(B, T, D)


# speedup vs baseline: 3.8671x; 1.0227x over previous
import jax
import jax.numpy as jnp
from jax import lax
from jax.experimental import pallas as pl
from jax.experimental.pallas import tpu as pltpu
from jax.experimental.pallas import tpu_sc as plsc

NC = 2
NS = 16
NW = NC * NS
LANES = 16
NB = 4
CS = 2


def kernel(X, table, pe):
    B, T = X.shape
    V, D = table.shape
    N = B * T
    scale_val = float(D) ** 0.5

    s_per_w = B // NW
    n_batches = s_per_w // (CS * NB)
    C = CS * T

    xflat = X.reshape(N)
    pe_t = pe[:T]

    mesh = plsc.VectorSubcoreMesh(
        core_axis_name="c", subcore_axis_name="s",
        num_cores=NC, num_subcores=NS)

    @pl.kernel(
        out_type=jax.ShapeDtypeStruct((B, T, D), jnp.float32),
        mesh=mesh,
        scratch_types=[
            pltpu.VMEM((NB * C,), jnp.int32),
            pltpu.VMEM((NB, CS, T, D), jnp.float32),
            pltpu.VMEM((T, D), jnp.float32),
            pltpu.SemaphoreType.DMA((NB, CS)),
            pltpu.SemaphoreType.DMA((NB,)),
        ],
        compiler_params=pltpu.CompilerParams(
            needs_layout_passes=False, use_tc_tiling_on_sc=False),
    )
    def emb_kernel(x_hbm, tbl_hbm, pe_hbm, out_hbm,
                   idx_v, rows_v, pe_v, gsem, ssem):
        wid = lax.axis_index("s") * NC + lax.axis_index("c")
        seq_base = wid * s_per_w
        pltpu.sync_copy(pe_hbm, pe_v)

        def batch_body(g, carry):
            bseq = seq_base + g * (NB * CS)
            pltpu.sync_copy(x_hbm.at[pl.ds(bseq * T, NB * C)], idx_v)

            for i in range(NB):
                @pl.when(g > 0)
                def _():
                    pltpu.make_async_copy(
                        rows_v.at[i], out_hbm.at[pl.ds(bseq, CS)],
                        ssem.at[i]).wait()
                for cs in range(CS):
                    pltpu.make_async_copy(
                        tbl_hbm.at[idx_v.at[pl.ds((i * CS + cs) * T, T)]],
                        rows_v.at[i, cs], gsem.at[i, cs]).start()

            for i in range(NB):
                for cs in range(CS):
                    pltpu.make_async_copy(
                        tbl_hbm.at[idx_v.at[pl.ds((i * CS + cs) * T, T)]],
                        rows_v.at[i, cs], gsem.at[i, cs]).wait()

                    @plsc.parallel_loop(0, T, 1, unroll=8)
                    def tok_body(t):
                        idx_splat = plsc.load_gather(
                            idx_v,
                            [jnp.full((LANES,), (i * CS + cs) * T + t,
                                      jnp.int32)])
                        s = jnp.where(idx_splat != 0,
                                      jnp.float32(scale_val),
                                      jnp.float32(0.0))
                        for d in range(D // LANES):
                            sl = pl.ds(d * LANES, LANES)
                            rows_v[i, cs, t, sl] = (
                                rows_v[i, cs, t, sl] * s + pe_v[t, sl])

                pltpu.make_async_copy(
                    rows_v.at[i], out_hbm.at[pl.ds(bseq + i * CS, CS)],
                    ssem.at[i]).start()
            return carry

        lax.fori_loop(0, n_batches, batch_body, None)

        last = seq_base + (n_batches - 1) * (NB * CS)
        for i in range(NB):
            pltpu.make_async_copy(
                rows_v.at[i], out_hbm.at[pl.ds(last + i * CS, CS)],
                ssem.at[i]).wait()

    return emb_kernel(xflat, table, pe_t)


# speedup vs baseline: 4.6404x; 1.2000x over previous
import jax
import jax.numpy as jnp
from jax import lax
from jax.experimental import pallas as pl
from jax.experimental.pallas import tpu as pltpu
from jax.experimental.pallas import tpu_sc as plsc

NC = 2
NS = 16
NW = NC * NS
LANES = 16
G = 16
DP = 128


def kernel(X, table, pe):
    B, T = X.shape
    V, D = table.shape
    N = B * T
    scale_val = float(D) ** 0.5

    s_per_w = B // NW
    n_groups = s_per_w // G

    xflat = X.reshape(N)
    tblp = jnp.pad(table, ((0, 0), (0, DP - D)))
    pe_flat = pe[:T].reshape(T * D)

    mesh = plsc.VectorSubcoreMesh(
        core_axis_name="c", subcore_axis_name="s",
        num_cores=NC, num_subcores=NS)

    @pl.kernel(
        out_type=jax.ShapeDtypeStruct((B, T, D), jnp.float32),
        mesh=mesh,
        scratch_types=[
            pltpu.VMEM((G * T,), jnp.int32),
            pltpu.VMEM((2, T, DP), jnp.float32),
            pltpu.VMEM((2, T, D), jnp.float32),
            pltpu.VMEM((T * D,), jnp.float32),
            pltpu.SemaphoreType.DMA((2,)),
            pltpu.SemaphoreType.DMA((2,)),
        ],
        compiler_params=pltpu.CompilerParams(
            needs_layout_passes=False, use_tc_tiling_on_sc=True),
    )
    def emb_kernel(x_hbm, tbl_hbm, pe_hbm, out_hbm,
                   idx_v, rows_v, o_v, pe_v, gsem, ssem):
        wid = lax.axis_index("s") * NC + lax.axis_index("c")
        seq_base = wid * s_per_w
        pltpu.sync_copy(pe_hbm, pe_v)

        def group_body(m, carry):
            gseq = seq_base + m * G
            pltpu.sync_copy(x_hbm.at[pl.ds(gseq * T, G * T)], idx_v)

            for k in range(2):
                pltpu.make_async_copy(
                    tbl_hbm.at[idx_v.at[pl.ds(k * T, T)]],
                    rows_v.at[k], gsem.at[k]).start()

            for k in range(G):
                b = k % 2
                pltpu.make_async_copy(
                    tbl_hbm.at[idx_v.at[pl.ds(k * T, T)]],
                    rows_v.at[b], gsem.at[b]).wait()

                def wait_store():
                    pltpu.make_async_copy(
                        o_v.at[b], out_hbm.at[gseq], ssem.at[b]).wait()
                if k < 2:
                    @pl.when(m > 0)
                    def _():
                        wait_store()
                else:
                    wait_store()

                @plsc.parallel_loop(0, T, 1, unroll=8)
                def tok_body(t):
                    idx_splat = plsc.load_gather(
                        idx_v, [jnp.full((LANES,), k * T + t, jnp.int32)])
                    s = jnp.where(idx_splat != 0,
                                  jnp.float32(scale_val), jnp.float32(0.0))
                    for d in range(D // LANES):
                        sl = pl.ds(d * LANES, LANES)
                        o_v[b, t, sl] = (
                            rows_v[b, t, sl] * s
                            + pe_v[pl.ds(t * D + d * LANES, LANES)])

                pltpu.make_async_copy(
                    o_v.at[b], out_hbm.at[gseq + k], ssem.at[b]).start()

                if k + 2 < G:
                    pltpu.make_async_copy(
                        tbl_hbm.at[idx_v.at[pl.ds((k + 2) * T, T)]],
                        rows_v.at[b], gsem.at[b]).start()
            return carry

        lax.fori_loop(0, n_groups, group_body, None)

        last = seq_base + s_per_w - 2
        for k in range(2):
            pltpu.make_async_copy(
                o_v.at[k], out_hbm.at[last + k], ssem.at[k]).wait()

    return emb_kernel(xflat, tblp, pe_flat)
